# baseline (device time: 55308 ns/iter reference)
import jax
import jax.numpy as jnp
from jax import lax
from jax.experimental import pallas as pl
from jax.experimental.pallas import tpu as pltpu

N_DEV = 8
B = 2
SQ_LOC = 128
D_MODEL = 512
HQ = 32
DH = 64
D_FF = HQ * DH
CHUNK = D_FF // N_DEV
HG = HQ // N_DEV
SKV = 128
CW_HOPS = 4
CCW_HOPS = 3


def _f(t):
    return jnp.where(t < 4, t, 11 - t)


def kernel(x, Wq, K_ext, V_ext, Wo):
    def body(x_ref, wq_ref, k_ref, v_ref, wo_ref, out_ref,
             wq_full, wo_full, xb, k_cm, v_cm, q_cm, ctx_c, acc,
             cw_send, cw_recv, ccw_send, ccw_recv):
        me = lax.axis_index("i")
        cp = _f(me)
        nxt = _f(jnp.mod(cp + 1, N_DEV))
        prv = _f(jnp.mod(cp - 1, N_DEV))
        is_even = jnp.mod(me, 2) == 0

        wq_full[:, pl.ds(me * CHUNK, CHUNK)] = wq_ref[...].astype(jnp.bfloat16)
        wo_full[pl.ds(me * CHUNK, CHUNK), :] = wo_ref[...].astype(jnp.bfloat16)

        barrier = pltpu.get_barrier_semaphore()
        for nbr in (nxt, prv):
            pl.semaphore_signal(
                barrier, inc=1, device_id=(nbr,),
                device_id_type=pl.DeviceIdType.MESH,
            )
        pl.semaphore_wait(barrier, 2)

        def start_pair(slot, sems_s, sems_r, h, target):
            rq = pltpu.make_async_remote_copy(
                src_ref=wq_full.at[:, pl.ds(slot * CHUNK, CHUNK)],
                dst_ref=wq_full.at[:, pl.ds(slot * CHUNK, CHUNK)],
                send_sem=sems_s.at[0, h],
                recv_sem=sems_r.at[0, h],
                device_id=(target,),
                device_id_type=pl.DeviceIdType.MESH,
            )
            ro = pltpu.make_async_remote_copy(
                src_ref=wo_full.at[pl.ds(slot * CHUNK, CHUNK), :],
                dst_ref=wo_full.at[pl.ds(slot * CHUNK, CHUNK), :],
                send_sem=sems_s.at[1, h],
                recv_sem=sems_r.at[1, h],
                device_id=(target,),
                device_id_type=pl.DeviceIdType.MESH,
            )
            rq.start()
            ro.start()
            return rq, ro

        def start_cw(h):
            return start_pair(_f(jnp.mod(cp - h, N_DEV)), cw_send, cw_recv,
                              h, nxt)

        def start_ccw(h):
            return start_pair(_f(jnp.mod(cp + h, N_DEV)), ccw_send, ccw_recv,
                              h, prv)

        def compute_chunk(o, first):
            wq_c = wq_full[:, pl.ds(o * CHUNK, CHUNK)]
            q_c = jnp.dot(xb[...], wq_c,
                          preferred_element_type=jnp.float32)
            q_c = q_c.astype(jnp.bfloat16)
            for b in range(B):
                for hh in range(HG):
                    q_cm[b * HG + hh] = q_c[b * SQ_LOC:(b + 1) * SQ_LOC,
                                            hh * DH:(hh + 1) * DH]
            qv = q_cm[...].reshape(B * HG * 2, 64, DH)
            kv = k_cm[pl.ds(o, 1)].reshape(B * HG * 2, 64, DH)
            s = lax.dot_general(
                qv, kv,
                dimension_numbers=(((2,), (2,)), ((0,), (0,))),
                preferred_element_type=jnp.float32,
            ) * 0.125
            m = jnp.max(s, axis=-1, keepdims=True)
            w = jnp.exp(s - m)
            wsum = jnp.sum(w, axis=-1, keepdims=True)
            w = (w / wsum).astype(jnp.bfloat16)
            ctx = lax.dot_general(
                w, v_cm[pl.ds(o, 1)].reshape(B * HG * 2, 64, DH),
                dimension_numbers=(((2,), (1,)), ((0,), (0,))),
                preferred_element_type=jnp.float32,
            ).reshape(B * HG, SQ_LOC, DH)
            for b in range(B):
                for hh in range(HG):
                    ctx_c[pl.ds(b * SQ_LOC, SQ_LOC), pl.ds(hh * DH, DH)] = (
                        ctx[b * HG + hh].astype(jnp.bfloat16))
            part = jnp.dot(ctx_c[...], wo_full[pl.ds(o * CHUNK, CHUNK), :],
                           preferred_element_type=jnp.float32)
            if first:
                acc[...] = part
            else:
                acc[...] = acc[...] + part

        cw = start_cw(0)
        ccw = start_ccw(0)

        @pl.when(is_even)
        def _():
            xb[...] = x_ref[...].reshape(B * SQ_LOC, D_MODEL).astype(
                jnp.bfloat16)
            for g in range(N_DEV):
                for b in range(B):
                    for hh in range(HG):
                        k_cm[g, b * HG + hh] = k_ref[
                            b, :, g * HG + hh, :].astype(jnp.bfloat16)
                        v_cm[g, b * HG + hh] = v_ref[
                            b, :, g * HG + hh, :].astype(jnp.bfloat16)
            compute_chunk(me, first=True)

        for r in cw + ccw:
            r.wait()

        for h in range(1, CW_HOPS):
            cw = start_cw(h)
            ccw = start_ccw(h) if h < CCW_HOPS else None

            @pl.when(is_even)
            def _(h=h):
                compute_chunk(_f(jnp.mod(cp - h, N_DEV)), first=False)
                compute_chunk(_f(jnp.mod(cp + h, N_DEV)), first=False)

            for r in cw:
                r.wait()
            if ccw is not None:
                for r in ccw:
                    r.wait()

        @pl.when(is_even)
        def _():
            compute_chunk(_f(jnp.mod(cp - 4, N_DEV)), first=False)
            out_ref[...] = acc[...].reshape(B, SQ_LOC, D_MODEL)

        @pl.when(jnp.logical_not(is_even))
        def _():
            out_ref[...] = jnp.zeros((B, SQ_LOC, D_MODEL), jnp.float32)

    return pl.pallas_call(
        body,
        out_shape=jax.ShapeDtypeStruct((B, SQ_LOC, D_MODEL), jnp.float32),
        in_specs=[pl.BlockSpec(memory_space=pltpu.VMEM)] * 5,
        out_specs=pl.BlockSpec(memory_space=pltpu.VMEM),
        scratch_shapes=[
            pltpu.VMEM((D_MODEL, D_FF), jnp.bfloat16),
            pltpu.VMEM((D_FF, D_MODEL), jnp.bfloat16),
            pltpu.VMEM((B * SQ_LOC, D_MODEL), jnp.bfloat16),
            pltpu.VMEM((N_DEV, B * HG, SKV, DH), jnp.bfloat16),
            pltpu.VMEM((N_DEV, B * HG, SKV, DH), jnp.bfloat16),
            pltpu.VMEM((B * HG, SQ_LOC, DH), jnp.bfloat16),
            pltpu.VMEM((B * SQ_LOC, HG * DH), jnp.bfloat16),
            pltpu.VMEM((B * SQ_LOC, D_MODEL), jnp.float32),
            pltpu.SemaphoreType.DMA((2, CW_HOPS)),
            pltpu.SemaphoreType.DMA((2, CW_HOPS)),
            pltpu.SemaphoreType.DMA((2, CCW_HOPS)),
            pltpu.SemaphoreType.DMA((2, CCW_HOPS)),
        ],
        compiler_params=pltpu.CompilerParams(collective_id=0),
    )(x, Wq, K_ext, V_ext, Wo)
